# baseline (speedup 1.0000x reference)
import jax
import jax.numpy as jnp
from jax.experimental import pallas as pl
from jax.experimental.pallas import tpu as pltpu

_T = 2000
_K = 20


def _stream(z_ref, w1_ref, b1t_ref, m2_ref, b2_ref):
    nb = z_ref.shape[0]
    h = jnp.concatenate(
        [jax.lax.dot_general(w1_ref[...], z_ref[b, 0],
                             (((0,), (1,)), ((), ())),
                             preferred_element_type=jnp.float32)
         for b in range(nb)], axis=0)
    h = jnp.tanh(h + b1t_ref[...])
    return jnp.tanh(
        jax.lax.dot_general(m2_ref[...], h, (((1,), (0,)), ((), ())),
                            preferred_element_type=jnp.float32)
        + b2_ref[...])


def _fused(zx_ref, ze_ref, wx1_ref, bx1t_ref, m2x_ref, bx2_ref,
           we1_ref, be1t_ref, m2e_ref, be2_ref, yp_ref,
           y_ref, xs_ref, es_ref, acc_ref):
    p = pl.program_id(0)
    i = pl.program_id(1)
    nb = 4
    f32 = jnp.float32

    @pl.when(p == 0)
    def _phase0():
        x4 = _stream(zx_ref, wx1_ref, bx1t_ref, m2x_ref, bx2_ref)
        e4 = _stream(ze_ref, we1_ref, be1t_ref, m2e_ref, be2_ref)
        xs_ref[i] = x4
        es_ref[i] = e4
        first = i == 0
        for b in range(nb):
            mnb = jnp.min(x4[b:b + 1, :])
            mxb = jnp.max(x4[b:b + 1, :])
            acc_ref[0, b] = jnp.where(first, mnb,
                                      jnp.minimum(acc_ref[0, b], mnb))
            acc_ref[0, nb + b] = jnp.where(first, mxb,
                                           jnp.maximum(acc_ref[0, nb + b],
                                                       mxb))
        se = jnp.sum(e4)
        sq = jnp.sum(e4 * e4)
        acc_ref[0, 2 * nb] = jnp.where(first, se, acc_ref[0, 2 * nb] + se)
        acc_ref[0, 2 * nb + 1] = jnp.where(first, sq,
                                           acc_ref[0, 2 * nb + 1] + sq)

    @pl.when(p == 1)
    def _phase1():
        T = xs_ref.shape[2]
        C = xs_ref.shape[0]
        ntot = f32(nb * C * T)
        S = acc_ref[0, 2 * nb]
        S2 = acc_ref[0, 2 * nb + 1]
        mean = S / ntot
        std = jnp.sqrt((S2 - S * S / ntot) / (ntot - f32(1.0)))
        alpha = f32(0.1) / std
        beta = -f32(0.1) * mean / std

        mn_col = jnp.concatenate(
            [jnp.full((1, 1), acc_ref[0, b], f32) for b in range(nb)], axis=0)
        mx_col = jnp.concatenate(
            [jnp.full((1, 1), acc_ref[0, nb + b], f32) for b in range(nb)],
            axis=0)
        d_col = mx_col - mn_col
        inv_d = f32(1.0) / d_col

        x4 = xs_ref[i]
        e4 = es_ref[i]
        h = 1.0 / (_K - 1)
        xpv = [s * h for s in range(_K)]
        y = yp_ref[:, 0:1] + (beta + alpha * e4)
        for s in range(_K - 1):
            ws = f32(xpv[s + 1] - xpv[s])
            a_col = mn_col + f32(xpv[s]) * d_col
            w_col = ws * d_col
            s_col = ((yp_ref[:, s + 1:s + 2] - yp_ref[:, s:s + 1])
                     / (ws + f32(1e-7))) * inv_d
            t = x4 - a_col
            t = jnp.maximum(jnp.minimum(t, w_col), f32(0.0))
            y = y + s_col * t
        y_ref[0] = y.T


def kernel(zf, zx, ze, Wf1, bf1, Wf2, bf2, Wx1, bx1, Wx2, bx2,
           We1, be1, We2, be2):
    B, N, L = zx.shape
    T = _T
    C = N // T
    f32 = jnp.float32

    pts = jnp.tanh(jnp.tanh(zf @ Wf1 + bf1) @ Wf2 + bf2)
    K = pts.shape[1]
    dirs = jax.random.randint(jax.random.key(42), (B,), 0, 2).astype(bool)
    srt = jnp.sort(pts, axis=1)
    yp = jnp.where(dirs[:, None], srt, srt[:, ::-1])

    zx4 = zx.reshape(B, C, T, L)
    ze4 = ze.reshape(B, C, T, L)
    bx1t = jnp.tile(bx1, B).reshape(B * L, 1).astype(f32)
    be1t = jnp.tile(be1, B).reshape(B * L, 1).astype(f32)
    eye = jnp.eye(B, dtype=f32)
    m2x = jnp.kron(eye, Wx2[:, 0][None, :])
    m2e = jnp.kron(eye, We2[:, 0][None, :])
    bx2c = bx2.reshape(1, 1)
    be2c = be2.reshape(1, 1)

    full = lambda shp: pl.BlockSpec(shp, lambda p, i: (0,) * len(shp))
    chunk_spec = pl.BlockSpec((B, 1, T, L),
                              lambda p, i: (0, i * (1 - p), 0, 0))

    ybuf = pl.pallas_call(
        _fused,
        grid=(2, C),
        in_specs=[
            chunk_spec, chunk_spec,
            full((L, L)), full((B * L, 1)), full((B, B * L)), full((1, 1)),
            full((L, L)), full((B * L, 1)), full((B, B * L)), full((1, 1)),
            full((B, K)),
        ],
        out_specs=pl.BlockSpec((1, T, B), lambda p, i: (i * p, 0, 0)),
        out_shape=jax.ShapeDtypeStruct((C, T, B), f32),
        scratch_shapes=[
            pltpu.VMEM((C, B, T), f32),
            pltpu.VMEM((C, B, T), f32),
            pltpu.SMEM((1, 16), f32),
        ],
    )(zx4, ze4, Wx1, bx1t, m2x, bx2c, We1, be1t, m2e, be2c, yp)

    return ybuf.reshape(N, B)


# speedup vs baseline: 1.1709x; 1.1709x over previous
import jax
import jax.numpy as jnp
from jax.experimental import pallas as pl
from jax.experimental.pallas import tpu as pltpu

_T = 2000
_K = 20


def _stream(z_ref, w1_ref, b1t_ref, m2_ref, b2_ref):
    nb = z_ref.shape[0]
    h = jnp.concatenate(
        [jax.lax.dot_general(w1_ref[...], z_ref[b, 0],
                             (((0,), (1,)), ((), ())),
                             preferred_element_type=jnp.float32)
         for b in range(nb)], axis=0)
    h = jnp.tanh(h + b1t_ref[...])
    return jnp.tanh(
        jax.lax.dot_general(m2_ref[...], h, (((1,), (0,)), ((), ())),
                            preferred_element_type=jnp.float32)
        + b2_ref[...])


def _fused(zx_ref, ze_ref, wx1_ref, bx1t_ref, m2x_ref, bx2_ref,
           we1_ref, be1t_ref, m2e_ref, be2_ref, yp_ref,
           y_ref, xs_ref, es_ref, acc_ref):
    p = pl.program_id(0)
    i = pl.program_id(1)
    nb = 4
    f32 = jnp.float32

    @pl.when(p == 0)
    def _phase0():
        x4 = _stream(zx_ref, wx1_ref, bx1t_ref, m2x_ref, bx2_ref)
        e4 = _stream(ze_ref, we1_ref, be1t_ref, m2e_ref, be2_ref)
        xs_ref[i] = x4
        es_ref[i] = e4
        first = i == 0
        for b in range(nb):
            mnb = jnp.min(x4[b:b + 1, :])
            mxb = jnp.max(x4[b:b + 1, :])
            acc_ref[0, b] = jnp.where(first, mnb,
                                      jnp.minimum(acc_ref[0, b], mnb))
            acc_ref[0, nb + b] = jnp.where(first, mxb,
                                           jnp.maximum(acc_ref[0, nb + b],
                                                       mxb))
        se = jnp.sum(e4)
        sq = jnp.sum(e4 * e4)
        acc_ref[0, 2 * nb] = jnp.where(first, se, acc_ref[0, 2 * nb] + se)
        acc_ref[0, 2 * nb + 1] = jnp.where(first, sq,
                                           acc_ref[0, 2 * nb + 1] + sq)

    @pl.when(p == 1)
    def _phase1():
        T = xs_ref.shape[2]
        C = xs_ref.shape[0]
        ntot = f32(nb * C * T)
        S = acc_ref[0, 2 * nb]
        S2 = acc_ref[0, 2 * nb + 1]
        mean = S / ntot
        std = jnp.sqrt((S2 - S * S / ntot) / (ntot - f32(1.0)))
        alpha = f32(0.1) / std
        beta = -f32(0.1) * mean / std

        mn_col = jnp.concatenate(
            [jnp.full((1, 1), acc_ref[0, b], f32) for b in range(nb)], axis=0)
        mx_col = jnp.concatenate(
            [jnp.full((1, 1), acc_ref[0, nb + b], f32) for b in range(nb)],
            axis=0)
        d_col = mx_col - mn_col
        inv_d = f32(1.0) / d_col

        x4 = xs_ref[i]
        e4 = es_ref[i]
        h = 1.0 / (_K - 1)
        xpv = [s * h for s in range(_K)]
        y = yp_ref[:, 0:1] + (beta + alpha * e4)
        for s in range(_K - 1):
            ws = f32(xpv[s + 1] - xpv[s])
            a_col = mn_col + f32(xpv[s]) * d_col
            w_col = ws * d_col
            s_col = ((yp_ref[:, s + 1:s + 2] - yp_ref[:, s:s + 1])
                     / (ws + f32(1e-7))) * inv_d
            t = x4 - a_col
            t = jnp.maximum(jnp.minimum(t, w_col), f32(0.0))
            y = y + s_col * t
        y_ref[0] = y


def kernel(zf, zx, ze, Wf1, bf1, Wf2, bf2, Wx1, bx1, Wx2, bx2,
           We1, be1, We2, be2):
    B, N, L = zx.shape
    T = _T
    C = N // T
    f32 = jnp.float32

    pts = jnp.tanh(jnp.tanh(zf @ Wf1 + bf1) @ Wf2 + bf2)
    K = pts.shape[1]
    dirs = jax.random.randint(jax.random.key(42), (B,), 0, 2).astype(bool)
    srt = jnp.sort(pts, axis=1)
    yp = jnp.where(dirs[:, None], srt, srt[:, ::-1])

    zx4 = zx.reshape(B, C, T, L)
    ze4 = ze.reshape(B, C, T, L)
    bx1t = jnp.tile(bx1, B).reshape(B * L, 1).astype(f32)
    be1t = jnp.tile(be1, B).reshape(B * L, 1).astype(f32)
    eye = jnp.eye(B, dtype=f32)
    m2x = jnp.kron(eye, Wx2[:, 0][None, :])
    m2e = jnp.kron(eye, We2[:, 0][None, :])
    bx2c = bx2.reshape(1, 1)
    be2c = be2.reshape(1, 1)

    full = lambda shp: pl.BlockSpec(shp, lambda p, i: (0,) * len(shp))
    chunk_spec = pl.BlockSpec((B, 1, T, L),
                              lambda p, i: (0, i * (1 - p), 0, 0))

    ybuf = pl.pallas_call(
        _fused,
        grid=(2, C),
        in_specs=[
            chunk_spec, chunk_spec,
            full((L, L)), full((B * L, 1)), full((B, B * L)), full((1, 1)),
            full((L, L)), full((B * L, 1)), full((B, B * L)), full((1, 1)),
            full((B, K)),
        ],
        out_specs=pl.BlockSpec((1, B, T), lambda p, i: (i * p, 0, 0)),
        out_shape=jax.ShapeDtypeStruct((C, B, T), f32),
        scratch_shapes=[
            pltpu.VMEM((C, B, T), f32),
            pltpu.VMEM((C, B, T), f32),
            pltpu.SMEM((1, 16), f32),
        ],
    )(zx4, ze4, Wx1, bx1t, m2x, bx2c, We1, be1t, m2e, be2c, yp)

    return ybuf.transpose(0, 2, 1).reshape(N, B)
